# Initial kernel scaffold; baseline (speedup 1.0000x reference)
import jax
import jax.numpy as jnp
from jax.experimental import pallas as pl


def kernel(x, edge_index, edge_type, W1_0, b1_0, g_0, be_0, W2_0, b2_0, bias_0, W1_1, b1_1, g_1, be_1, W2_1, b2_1, bias_1):
    raise NotImplementedError("write your pallas kernel here")


# speedup vs baseline: 8.3200x; 8.3200x over previous
import functools

import jax
import jax.numpy as jnp
from jax import lax
from jax.experimental import pallas as pl
from jax.experimental.pallas import tpu as pltpu
from jax.experimental.pallas import tpu_sc as plsc

N = 10000
E = 320000
DIN = 128
HID = 64
DOUT = 128
NREL = 4

NSUB = 16
ROWS = NREL * N + 64
C = 128
CHUNKS = 160
E_PAD = NSUB * CHUNKS * C
EIDX_ROWS = E_PAD // 128
NBUF = 8
GROUPS = CHUNKS // NBUF
HGROUPS = GROUPS // 2
RPT = ROWS // NSUB

BLK = 1000
NBLK = N // BLK


def _dot(a, b):
    return jnp.dot(a, b, preferred_element_type=jnp.float32,
                   precision=lax.Precision.HIGHEST)


def _idx_body(src_ref, dst_ref, typ_ref, g2_ref, s2_ref):
    typ = typ_ref[...]
    g = typ * N + src_ref[...]
    g2_ref[0] = g
    g2_ref[1] = g + ROWS
    s2_ref[...] = typ * N + dst_ref[...]


def _pre_body(x_ref, w1_ref, y2_ref, rp_ref, rst_ref):
    i = pl.program_id(0)
    xb = x_ref[...]
    for r in range(NREL):
        y = _dot(xb, w1_ref[r])
        for c in range(2):
            y2_ref[pl.ds(c * ROWS + r * N + i * BLK, BLK), :] = y[:, 32 * c:32 * c + 32]
    rp = _dot(xb, w1_ref[NREL])
    rp_ref[...] = rp
    s0 = jnp.sum(rp, axis=0, keepdims=True)
    s1 = jnp.sum(rp * rp, axis=0, keepdims=True)
    new = jnp.concatenate([s0, s1], axis=0)

    @pl.when(i == 0)
    def _():
        rst_ref[...] = jnp.zeros_like(rst_ref)

    rst_ref[...] += new


def _stats_body(h_ref, st_ref):
    b = pl.program_id(2)
    xb = h_ref[0]
    s0 = jnp.sum(xb, axis=0, keepdims=True)
    s1 = jnp.sum(xb * xb, axis=0, keepdims=True)
    new = jnp.concatenate([s0, s1], axis=0).reshape(1, 1, 2, 32)

    @pl.when(b == 0)
    def _():
        st_ref[...] = jnp.zeros_like(st_ref)

    st_ref[...] += new


def _bn_relu(hb, m, inv, gg, bb):
    return jax.nn.relu((hb - m) * inv * gg + bb)


def _make_apply_body(dout, final_relu):
    def body(h_ref, st_ref, rp_ref, rst_ref, g_ref, be_ref, w2_ref, b2_ref,
             bias_ref, out_ref):
        i = pl.program_id(0)
        acc = jnp.zeros((BLK, dout), jnp.float32)
        for r in range(NREL):
            for ch in range(2):
                cs = 32 * ch
                m = st_ref[ch, r, 0:1, :] * (1.0 / N)
                ex2 = st_ref[ch, r, 1:2, :] * (1.0 / N)
                inv = lax.rsqrt(ex2 - m * m + 1e-5)
                hb = h_ref[ch, pl.ds(r * N + i * BLK, BLK), :]
                a = _bn_relu(hb, m, inv, g_ref[r:r + 1, cs:cs + 32],
                             be_ref[r:r + 1, cs:cs + 32])
                acc = acc + _dot(a, w2_ref[r, cs:cs + 32, :])
            acc = acc + b2_ref[r:r + 1, :]
        m = rst_ref[0:1, :] * (1.0 / N)
        ex2 = rst_ref[1:2, :] * (1.0 / N)
        inv = lax.rsqrt(ex2 - m * m + 1e-5)
        a = _bn_relu(rp_ref[...], m, inv, g_ref[NREL:NREL + 1, :],
                     be_ref[NREL:NREL + 1, :])
        acc = acc + _dot(a, w2_ref[NREL]) + b2_ref[NREL:NREL + 1, :] + bias_ref[...]
        out_ref[...] = jax.nn.relu(acc) if final_relu else acc
    return body


def _tc_idx(srcp, dstp, typp):
    blk = EIDX_ROWS // 10
    return pl.pallas_call(
        _idx_body,
        grid=(10,),
        in_specs=[pl.BlockSpec((blk, 128), lambda i: (i, 0))] * 3,
        out_specs=[pl.BlockSpec((2, blk, 128), lambda i: (0, i, 0)),
                   pl.BlockSpec((blk, 128), lambda i: (i, 0))],
        out_shape=[jax.ShapeDtypeStruct((2, EIDX_ROWS, 128), jnp.int32),
                   jax.ShapeDtypeStruct((EIDX_ROWS, 128), jnp.int32)],
    )(srcp, dstp, typp)


def _tc_pre(x, w1):
    din = x.shape[1]
    return pl.pallas_call(
        _pre_body,
        grid=(NBLK,),
        in_specs=[pl.BlockSpec((BLK, din), lambda i: (i, 0)),
                  pl.BlockSpec((NREL + 1, din, HID), lambda i: (0, 0, 0))],
        out_specs=[pl.BlockSpec((2 * ROWS, 32), lambda i: (0, 0)),
                   pl.BlockSpec((BLK, HID), lambda i: (i, 0)),
                   pl.BlockSpec((2, HID), lambda i: (0, 0))],
        out_shape=[jax.ShapeDtypeStruct((2 * ROWS, 32), jnp.float32),
                   jax.ShapeDtypeStruct((N, HID), jnp.float32),
                   jax.ShapeDtypeStruct((2, HID), jnp.float32)],
    )(x, w1)


def _tc_stats(h):
    return pl.pallas_call(
        _stats_body,
        grid=(2, NREL, NBLK),
        in_specs=[pl.BlockSpec((1, BLK, 32), lambda c, r, b: (c, r * NBLK + b, 0))],
        out_specs=pl.BlockSpec((1, 1, 2, 32), lambda c, r, b: (c, r, 0, 0)),
        out_shape=jax.ShapeDtypeStruct((2, NREL, 2, 32), jnp.float32),
    )(h)


def _tc_apply(h, st, rp, rst, g, be, w2, b2, bias, dout, final_relu):
    return pl.pallas_call(
        _make_apply_body(dout, final_relu),
        grid=(NBLK,),
        in_specs=[pl.BlockSpec((2, ROWS, 32), lambda i: (0, 0, 0)),
                  pl.BlockSpec((2, NREL, 2, 32), lambda i: (0, 0, 0, 0)),
                  pl.BlockSpec((BLK, HID), lambda i: (i, 0)),
                  pl.BlockSpec((2, HID), lambda i: (0, 0)),
                  pl.BlockSpec((NREL + 1, HID), lambda i: (0, 0)),
                  pl.BlockSpec((NREL + 1, HID), lambda i: (0, 0)),
                  pl.BlockSpec((NREL + 1, HID, dout), lambda i: (0, 0, 0)),
                  pl.BlockSpec((NREL + 1, dout), lambda i: (0, 0)),
                  pl.BlockSpec((1, dout), lambda i: (0, 0))],
        out_specs=pl.BlockSpec((BLK, dout), lambda i: (i, 0)),
        out_shape=jax.ShapeDtypeStruct((N, dout), jnp.float32),
    )(h, st, rp, rst, g, be, w2, b2, bias)


def _edge_body(y2, g2, s2, z, out, idxg, idxs, rows, acc, gsems, isems):
    cid = lax.axis_index("c")
    sid = lax.axis_index("s")
    base = sid * CHUNKS
    pltpu.sync_copy(z.at[pl.ds(sid * RPT, RPT)], acc.at[pl.ds(sid * RPT, RPT)])
    for p in (0, 1):
        pltpu.async_copy(g2.at[cid, pl.ds(base + p * NBUF, NBUF)], idxg.at[p],
                         isems.at[2 * p])
        pltpu.async_copy(s2.at[pl.ds(base + p * NBUF, NBUF)], idxs.at[p],
                         isems.at[2 * p + 1])
    plsc.subcore_barrier()

    @pl.loop(0, HGROUPS)
    def _(gi2):
        for p in (0, 1):
            gi = gi2 * 2 + p
            pltpu.make_async_copy(g2.at[cid, pl.ds(base, NBUF)], idxg.at[p],
                                  isems.at[2 * p]).wait()
            pltpu.make_async_copy(s2.at[pl.ds(base, NBUF)], idxs.at[p],
                                  isems.at[2 * p + 1]).wait()
            descs = [pltpu.async_copy(y2.at[idxg.at[p, b]], rows.at[b], gsems.at[b])
                     for b in range(NBUF)]
            for b in range(NBUF):
                descs[b].wait()
                pltpu.sync_copy(rows.at[b], acc.at[idxs.at[p, b]], add=True)

            @pl.when(gi2 < HGROUPS - 1)
            def _():
                off = base + (gi + 2) * NBUF
                pltpu.async_copy(g2.at[cid, pl.ds(off, NBUF)], idxg.at[p],
                                 isems.at[2 * p])
                pltpu.async_copy(s2.at[pl.ds(off, NBUF)], idxs.at[p],
                                 isems.at[2 * p + 1])

    plsc.subcore_barrier()
    pltpu.sync_copy(acc.at[pl.ds(sid * RPT, RPT)], out.at[cid, pl.ds(sid * RPT, RPT)])


def _edge_pass(y2, g2, s2, z):
    mesh = plsc.VectorSubcoreMesh(core_axis_name="c", subcore_axis_name="s")
    f = pl.kernel(
        _edge_body,
        out_type=jax.ShapeDtypeStruct((2, ROWS, 32), jnp.float32),
        mesh=mesh,
        scratch_types=[
            pltpu.VMEM((2, NBUF, C), jnp.int32),
            pltpu.VMEM((2, NBUF, C), jnp.int32),
            pltpu.VMEM((NBUF, C, 32), jnp.float32),
            pltpu.VMEM_SHARED((ROWS, 32), jnp.float32),
            pltpu.SemaphoreType.DMA((NBUF,)),
            pltpu.SemaphoreType.DMA((4,)),
        ],
        compiler_params=pltpu.CompilerParams(use_tc_tiling_on_sc=False),
    )
    return f(y2, g2, s2, z)


def kernel(x, edge_index, edge_type,
           W1_0, b1_0, g_0, be_0, W2_0, b2_0, bias_0,
           W1_1, b1_1, g_1, be_1, W2_1, b2_1, bias_1):
    pad = E_PAD - E
    zpad = jnp.zeros((pad,), jnp.int32)
    srcp = jnp.concatenate([edge_index[0], zpad]).reshape(EIDX_ROWS, 128)
    dstp = jnp.concatenate([edge_index[1], zpad]).reshape(EIDX_ROWS, 128)
    typp = jnp.concatenate([edge_type, jnp.full((pad,), NREL, jnp.int32)]
                           ).reshape(EIDX_ROWS, 128)
    z = jnp.zeros((ROWS, 32), jnp.float32)

    g2, s2 = _tc_idx(srcp, dstp, typp)

    y2_0, rp0, rst0 = _tc_pre(x, W1_0)
    h0 = _edge_pass(y2_0, g2, s2, z)
    st0 = _tc_stats(h0)
    h1 = _tc_apply(h0, st0, rp0, rst0, g_0, be_0, W2_0, b2_0,
                   bias_0.reshape(1, HID), HID, True)

    y2_1, rp1, rst1 = _tc_pre(h1, W1_1)
    h1acc = _edge_pass(y2_1, g2, s2, z)
    st1 = _tc_stats(h1acc)
    out = _tc_apply(h1acc, st1, rp1, rst1, g_1, be_1, W2_1, b2_1,
                    bias_1.reshape(1, DOUT), DOUT, False)
    return out


# speedup vs baseline: 8.4839x; 1.0197x over previous
import functools

import jax
import jax.numpy as jnp
from jax import lax
from jax.experimental import pallas as pl
from jax.experimental.pallas import tpu as pltpu
from jax.experimental.pallas import tpu_sc as plsc

N = 10000
E = 320000
DIN = 128
HID = 64
DOUT = 128
NREL = 4

NSUB = 16
ROWS = NREL * N + 64
C = 128
CHUNKS = 160
E_PAD = NSUB * CHUNKS * C
EIDX_ROWS = E_PAD // 128
NBUF = 8
GROUPS = CHUNKS // NBUF
NSLOT = 4
RPT = ROWS // NSUB

BLK = 1000
NBLK = N // BLK


def _dot(a, b):
    return jnp.dot(a, b, preferred_element_type=jnp.float32,
                   precision=lax.Precision.HIGHEST)


def _idx_body(src_ref, dst_ref, typ_ref, g2_ref, s2_ref):
    typ = typ_ref[...]
    g = typ * N + src_ref[...]
    g2_ref[0] = g
    g2_ref[1] = g + ROWS
    s2_ref[...] = typ * N + dst_ref[...]


def _pre_body(x_ref, w1_ref, y2_ref, rp_ref, rst_ref):
    i = pl.program_id(0)
    xb = x_ref[...]
    for r in range(NREL):
        y = _dot(xb, w1_ref[r])
        for c in range(2):
            y2_ref[pl.ds(c * ROWS + r * N + i * BLK, BLK), :] = y[:, 32 * c:32 * c + 32]
    rp = _dot(xb, w1_ref[NREL])
    rp_ref[...] = rp
    s0 = jnp.sum(rp, axis=0, keepdims=True)
    s1 = jnp.sum(rp * rp, axis=0, keepdims=True)
    new = jnp.concatenate([s0, s1], axis=0)

    @pl.when(i == 0)
    def _():
        rst_ref[...] = jnp.zeros_like(rst_ref)

    rst_ref[...] += new


def _stats_body(h_ref, st_ref):
    b = pl.program_id(2)
    xb = h_ref[0]
    s0 = jnp.sum(xb, axis=0, keepdims=True)
    s1 = jnp.sum(xb * xb, axis=0, keepdims=True)
    new = jnp.concatenate([s0, s1], axis=0).reshape(1, 1, 2, 32)

    @pl.when(b == 0)
    def _():
        st_ref[...] = jnp.zeros_like(st_ref)

    st_ref[...] += new


def _bn_relu(hb, m, inv, gg, bb):
    return jax.nn.relu((hb - m) * inv * gg + bb)


def _make_apply_body(dout, final_relu):
    def body(h_ref, st_ref, rp_ref, rst_ref, g_ref, be_ref, w2_ref, b2_ref,
             bias_ref, out_ref):
        i = pl.program_id(0)
        acc = jnp.zeros((BLK, dout), jnp.float32)
        for r in range(NREL):
            for ch in range(2):
                cs = 32 * ch
                m = st_ref[ch, r, 0:1, :] * (1.0 / N)
                ex2 = st_ref[ch, r, 1:2, :] * (1.0 / N)
                inv = lax.rsqrt(ex2 - m * m + 1e-5)
                hb = h_ref[ch, pl.ds(r * N + i * BLK, BLK), :]
                a = _bn_relu(hb, m, inv, g_ref[r:r + 1, cs:cs + 32],
                             be_ref[r:r + 1, cs:cs + 32])
                acc = acc + _dot(a, w2_ref[r, cs:cs + 32, :])
            acc = acc + b2_ref[r:r + 1, :]
        m = rst_ref[0:1, :] * (1.0 / N)
        ex2 = rst_ref[1:2, :] * (1.0 / N)
        inv = lax.rsqrt(ex2 - m * m + 1e-5)
        a = _bn_relu(rp_ref[...], m, inv, g_ref[NREL:NREL + 1, :],
                     be_ref[NREL:NREL + 1, :])
        acc = acc + _dot(a, w2_ref[NREL]) + b2_ref[NREL:NREL + 1, :] + bias_ref[...]
        out_ref[...] = jax.nn.relu(acc) if final_relu else acc
    return body


def _tc_idx(srcp, dstp, typp):
    blk = EIDX_ROWS // 10
    return pl.pallas_call(
        _idx_body,
        grid=(10,),
        in_specs=[pl.BlockSpec((blk, 128), lambda i: (i, 0))] * 3,
        out_specs=[pl.BlockSpec((2, blk, 128), lambda i: (0, i, 0)),
                   pl.BlockSpec((blk, 128), lambda i: (i, 0))],
        out_shape=[jax.ShapeDtypeStruct((2, EIDX_ROWS, 128), jnp.int32),
                   jax.ShapeDtypeStruct((EIDX_ROWS, 128), jnp.int32)],
    )(srcp, dstp, typp)


def _tc_pre(x, w1):
    din = x.shape[1]
    return pl.pallas_call(
        _pre_body,
        grid=(NBLK,),
        in_specs=[pl.BlockSpec((BLK, din), lambda i: (i, 0)),
                  pl.BlockSpec((NREL + 1, din, HID), lambda i: (0, 0, 0))],
        out_specs=[pl.BlockSpec((2 * ROWS, 32), lambda i: (0, 0)),
                   pl.BlockSpec((BLK, HID), lambda i: (i, 0)),
                   pl.BlockSpec((2, HID), lambda i: (0, 0))],
        out_shape=[jax.ShapeDtypeStruct((2 * ROWS, 32), jnp.float32),
                   jax.ShapeDtypeStruct((N, HID), jnp.float32),
                   jax.ShapeDtypeStruct((2, HID), jnp.float32)],
    )(x, w1)


def _tc_stats(h):
    return pl.pallas_call(
        _stats_body,
        grid=(2, NREL, NBLK),
        in_specs=[pl.BlockSpec((1, BLK, 32), lambda c, r, b: (c, r * NBLK + b, 0))],
        out_specs=pl.BlockSpec((1, 1, 2, 32), lambda c, r, b: (c, r, 0, 0)),
        out_shape=jax.ShapeDtypeStruct((2, NREL, 2, 32), jnp.float32),
    )(h)


def _tc_apply(h, st, rp, rst, g, be, w2, b2, bias, dout, final_relu):
    return pl.pallas_call(
        _make_apply_body(dout, final_relu),
        grid=(NBLK,),
        in_specs=[pl.BlockSpec((2, ROWS, 32), lambda i: (0, 0, 0)),
                  pl.BlockSpec((2, NREL, 2, 32), lambda i: (0, 0, 0, 0)),
                  pl.BlockSpec((BLK, HID), lambda i: (i, 0)),
                  pl.BlockSpec((2, HID), lambda i: (0, 0)),
                  pl.BlockSpec((NREL + 1, HID), lambda i: (0, 0)),
                  pl.BlockSpec((NREL + 1, HID), lambda i: (0, 0)),
                  pl.BlockSpec((NREL + 1, HID, dout), lambda i: (0, 0, 0)),
                  pl.BlockSpec((NREL + 1, dout), lambda i: (0, 0)),
                  pl.BlockSpec((1, dout), lambda i: (0, 0))],
        out_specs=pl.BlockSpec((BLK, dout), lambda i: (i, 0)),
        out_shape=jax.ShapeDtypeStruct((N, dout), jnp.float32),
    )(h, st, rp, rst, g, be, w2, b2, bias)


def _edge_body(y2, g2, s2, z, out, idxg, idxs, rows, acc, gsems, ssems, isems):
    cid = lax.axis_index("c")
    sid = lax.axis_index("s")
    base = sid * CHUNKS
    pltpu.sync_copy(z.at[pl.ds(sid * RPT, RPT)], acc.at[pl.ds(sid * RPT, RPT)])
    for p in (0, 1):
        pltpu.async_copy(g2.at[cid, pl.ds(base + p * NBUF, NBUF)], idxg.at[p],
                         isems.at[2 * p])
        pltpu.async_copy(s2.at[pl.ds(base + p * NBUF, NBUF)], idxs.at[p],
                         isems.at[2 * p + 1])
    plsc.subcore_barrier()

    @pl.loop(0, GROUPS // NSLOT)
    def _(it):
        for p in range(NSLOT):
            gi = it * NSLOT + p
            pltpu.make_async_copy(g2.at[cid, pl.ds(base, NBUF)], idxg.at[p],
                                  isems.at[2 * p]).wait()
            pltpu.make_async_copy(s2.at[pl.ds(base, NBUF)], idxs.at[p],
                                  isems.at[2 * p + 1]).wait()
            for b in range(NBUF):
                if p == 0:
                    @pl.when(it > 0)
                    def _():
                        pltpu.make_async_copy(rows.at[b], acc.at[idxs.at[p, b]],
                                              ssems.at[b]).wait()
                else:
                    pltpu.make_async_copy(rows.at[b], acc.at[idxs.at[p, b]],
                                          ssems.at[b]).wait()
                pltpu.async_copy(y2.at[idxg.at[p, b]], rows.at[b], gsems.at[b])

            q = (p + 2) % NSLOT

            @pl.when(gi < GROUPS - 2)
            def _():
                off = base + (gi + 2) * NBUF
                pltpu.async_copy(g2.at[cid, pl.ds(off, NBUF)], idxg.at[q],
                                 isems.at[2 * q])
                pltpu.async_copy(s2.at[pl.ds(off, NBUF)], idxs.at[q],
                                 isems.at[2 * q + 1])
            for b in range(NBUF):
                pltpu.make_async_copy(y2.at[idxg.at[p, b]], rows.at[b],
                                      gsems.at[b]).wait()
                pltpu.async_copy(rows.at[b], acc.at[idxs.at[p, b]], ssems.at[b],
                                 add=True)

    for b in range(NBUF):
        pltpu.make_async_copy(rows.at[b], acc.at[idxs.at[NSLOT - 1, b]],
                              ssems.at[b]).wait()
    plsc.subcore_barrier()
    pltpu.sync_copy(acc.at[pl.ds(sid * RPT, RPT)], out.at[cid, pl.ds(sid * RPT, RPT)])


def _edge_pass(y2, g2, s2, z):
    mesh = plsc.VectorSubcoreMesh(core_axis_name="c", subcore_axis_name="s")
    f = pl.kernel(
        _edge_body,
        out_type=jax.ShapeDtypeStruct((2, ROWS, 32), jnp.float32),
        mesh=mesh,
        scratch_types=[
            pltpu.VMEM((NSLOT, NBUF, C), jnp.int32),
            pltpu.VMEM((NSLOT, NBUF, C), jnp.int32),
            pltpu.VMEM((NBUF, C, 32), jnp.float32),
            pltpu.VMEM_SHARED((ROWS, 32), jnp.float32),
            pltpu.SemaphoreType.DMA((NBUF,)),
            pltpu.SemaphoreType.DMA((NBUF,)),
            pltpu.SemaphoreType.DMA((2 * NSLOT,)),
        ],
        compiler_params=pltpu.CompilerParams(use_tc_tiling_on_sc=False),
    )
    return f(y2, g2, s2, z)


def kernel(x, edge_index, edge_type,
           W1_0, b1_0, g_0, be_0, W2_0, b2_0, bias_0,
           W1_1, b1_1, g_1, be_1, W2_1, b2_1, bias_1):
    pad = E_PAD - E
    zpad = jnp.zeros((pad,), jnp.int32)
    srcp = jnp.concatenate([edge_index[0], zpad]).reshape(EIDX_ROWS, 128)
    dstp = jnp.concatenate([edge_index[1], zpad]).reshape(EIDX_ROWS, 128)
    typp = jnp.concatenate([edge_type, jnp.full((pad,), NREL, jnp.int32)]
                           ).reshape(EIDX_ROWS, 128)
    z = jnp.zeros((ROWS, 32), jnp.float32)

    g2, s2 = _tc_idx(srcp, dstp, typp)

    y2_0, rp0, rst0 = _tc_pre(x, W1_0)
    h0 = _edge_pass(y2_0, g2, s2, z)
    st0 = _tc_stats(h0)
    h1 = _tc_apply(h0, st0, rp0, rst0, g_0, be_0, W2_0, b2_0,
                   bias_0.reshape(1, HID), HID, True)

    y2_1, rp1, rst1 = _tc_pre(h1, W1_1)
    h1acc = _edge_pass(y2_1, g2, s2, z)
    st1 = _tc_stats(h1acc)
    out = _tc_apply(h1acc, st1, rp1, rst1, g_1, be_1, W2_1, b2_1,
                    bias_1.reshape(1, DOUT), DOUT, False)
    return out


# speedup vs baseline: 9.5343x; 1.1238x over previous
import functools

import jax
import jax.numpy as jnp
from jax import lax
from jax.experimental import pallas as pl
from jax.experimental.pallas import tpu as pltpu
from jax.experimental.pallas import tpu_sc as plsc

N = 10000
E = 320000
DIN = 128
HID = 64
DOUT = 128
NREL = 4

NSUB = 16
ROWS = NREL * N + 64
C = 128
CHUNKS = 160
E_PAD = NSUB * CHUNKS * C
EIDX_ROWS = E_PAD // 128
NBUF = 8
GROUPS = CHUNKS // NBUF
NSLOT = 4
RPT = ROWS // NSUB
WROWS = NREL * N
WPT = WROWS // NSUB

BLK = 1000
NBLK = N // BLK


def _dot(a, b):
    return jnp.dot(a, b, preferred_element_type=jnp.float32,
                   precision=lax.Precision.HIGHEST)


def _pre0_body(x_ref, w1_ref, src_ref, dst_ref, typ_ref,
               y2_ref, rp_ref, rst_ref, g2_ref, s2_ref):
    i = pl.program_id(0)
    xb = x_ref[...]
    for r in range(NREL):
        y = _dot(xb, w1_ref[r])
        for c in range(2):
            y2_ref[c, r] = y[:, 32 * c:32 * c + 32]
    rp = _dot(xb, w1_ref[NREL])
    rp_ref[...] = rp
    s0 = jnp.sum(rp, axis=0, keepdims=True)
    s1 = jnp.sum(rp * rp, axis=0, keepdims=True)
    new = jnp.concatenate([s0, s1], axis=0)

    @pl.when(i == 0)
    def _():
        rst_ref[...] = jnp.zeros_like(rst_ref)

    rst_ref[...] += new
    typ = typ_ref[...]
    g0 = jnp.where(typ < NREL, typ * N + src_ref[...], 0)
    g2_ref[0] = g0
    g2_ref[1] = g0 + WROWS
    s2_ref[...] = typ * N + dst_ref[...]


def _tc_pre0(x, w1, srcp, dstp, typp):
    eblk = EIDX_ROWS // NBLK
    return pl.pallas_call(
        _pre0_body,
        grid=(NBLK,),
        in_specs=[pl.BlockSpec((BLK, DIN), lambda i: (i, 0)),
                  pl.BlockSpec((NREL + 1, DIN, HID), lambda i: (0, 0, 0)),
                  pl.BlockSpec((eblk, 128), lambda i: (i, 0)),
                  pl.BlockSpec((eblk, 128), lambda i: (i, 0)),
                  pl.BlockSpec((eblk, 128), lambda i: (i, 0))],
        out_specs=[pl.BlockSpec((2, NREL, BLK, 32), lambda i: (0, 0, i, 0)),
                   pl.BlockSpec((BLK, HID), lambda i: (i, 0)),
                   pl.BlockSpec((2, HID), lambda i: (0, 0)),
                   pl.BlockSpec((2, eblk, 128), lambda i: (0, i, 0)),
                   pl.BlockSpec((eblk, 128), lambda i: (i, 0))],
        out_shape=[jax.ShapeDtypeStruct((2, NREL, N, 32), jnp.float32),
                   jax.ShapeDtypeStruct((N, HID), jnp.float32),
                   jax.ShapeDtypeStruct((2, HID), jnp.float32),
                   jax.ShapeDtypeStruct((2, EIDX_ROWS, 128), jnp.int32),
                   jax.ShapeDtypeStruct((EIDX_ROWS, 128), jnp.int32)],
    )(x, w1, srcp, dstp, typp)


def _bn_relu(hb, m, inv, gg, bb):
    return jax.nn.relu((hb - m) * inv * gg + bb)


def _accum_stats(st_ref, h_ref):
    for ch in range(2):
        for r in range(NREL):
            hb = h_ref[ch, r]
            s0 = jnp.sum(hb, axis=0, keepdims=True)
            s1 = jnp.sum(hb * hb, axis=0, keepdims=True)
            st_ref[ch, r] = st_ref[ch, r] + jnp.concatenate([s0, s1], axis=0)


def _apply_block(h_ref, st_ref, rp_ref, rst_ref, g_ref, be_ref, w2_ref,
                 b2_ref, bias_ref, dout):
    acc = jnp.zeros((BLK, dout), jnp.float32)
    for r in range(NREL):
        for ch in range(2):
            cs = 32 * ch
            st = st_ref[ch, r]
            m = st[0:1, :] * (1.0 / N)
            ex2 = st[1:2, :] * (1.0 / N)
            inv = lax.rsqrt(ex2 - m * m + 1e-5)
            hb = h_ref[ch, r]
            a = _bn_relu(hb, m, inv, g_ref[r:r + 1, cs:cs + 32],
                         be_ref[r:r + 1, cs:cs + 32])
            acc = acc + _dot(a, w2_ref[r, cs:cs + 32, :])
        acc = acc + b2_ref[r:r + 1, :]
    m = rst_ref[0:1, :] * (1.0 / N)
    ex2 = rst_ref[1:2, :] * (1.0 / N)
    inv = lax.rsqrt(ex2 - m * m + 1e-5)
    a = _bn_relu(rp_ref[...], m, inv, g_ref[NREL:NREL + 1, :],
                 be_ref[NREL:NREL + 1, :])
    return acc + _dot(a, w2_ref[NREL]) + b2_ref[NREL:NREL + 1, :] + bias_ref[...]

def _stats_body(h_ref, st_ref):
    i = pl.program_id(0)

    @pl.when(i == 0)
    def _():
        st_ref[...] = jnp.zeros_like(st_ref)

    _accum_stats(st_ref, h_ref)


def _tc_stats(h):
    return pl.pallas_call(
        _stats_body,
        grid=(NBLK,),
        in_specs=[pl.BlockSpec((2, NREL, BLK, 32), lambda i: (0, 0, i, 0))],
        out_specs=pl.BlockSpec((2, NREL, 2, 32), lambda i: (0, 0, 0, 0)),
        out_shape=jax.ShapeDtypeStruct((2, NREL, 2, 32), jnp.float32),
    )(h)


def _post0_body(h_ref, st4_ref, rp_ref, rst_ref, g_ref, be_ref, w2_ref,
                b2_ref, bias_ref, w1n_ref, y2_ref, rp1_ref, rst1_ref):
    i = pl.program_id(0)
    h1 = jax.nn.relu(_apply_block(h_ref, st4_ref, rp_ref, rst_ref, g_ref,
                                  be_ref, w2_ref, b2_ref, bias_ref, HID))
    for r in range(NREL):
        y = _dot(h1, w1n_ref[r])
        for c in range(2):
            y2_ref[c, r] = y[:, 32 * c:32 * c + 32]
    rp1 = _dot(h1, w1n_ref[NREL])
    rp1_ref[...] = rp1
    s0 = jnp.sum(rp1, axis=0, keepdims=True)
    s1 = jnp.sum(rp1 * rp1, axis=0, keepdims=True)

    @pl.when(i == 0)
    def _():
        rst1_ref[...] = jnp.zeros_like(rst1_ref)

    rst1_ref[...] += jnp.concatenate([s0, s1], axis=0)


def _tc_post0(h, st4, rp, rst, g, be, w2, b2, bias, w1n):
    return pl.pallas_call(
        _post0_body,
        grid=(NBLK,),
        in_specs=[pl.BlockSpec((2, NREL, BLK, 32), lambda i: (0, 0, i, 0)),
                  pl.BlockSpec((2, NREL, 2, 32), lambda i: (0, 0, 0, 0)),
                  pl.BlockSpec((BLK, HID), lambda i: (i, 0)),
                  pl.BlockSpec((2, HID), lambda i: (0, 0)),
                  pl.BlockSpec((NREL + 1, HID), lambda i: (0, 0)),
                  pl.BlockSpec((NREL + 1, HID), lambda i: (0, 0)),
                  pl.BlockSpec((NREL + 1, HID, HID), lambda i: (0, 0, 0)),
                  pl.BlockSpec((NREL + 1, HID), lambda i: (0, 0)),
                  pl.BlockSpec((1, HID), lambda i: (0, 0)),
                  pl.BlockSpec((NREL + 1, HID, HID), lambda i: (0, 0, 0))],
        out_specs=[pl.BlockSpec((2, NREL, BLK, 32), lambda i: (0, 0, i, 0)),
                   pl.BlockSpec((BLK, HID), lambda i: (i, 0)),
                   pl.BlockSpec((2, HID), lambda i: (0, 0))],
        out_shape=[jax.ShapeDtypeStruct((2, NREL, N, 32), jnp.float32),
                   jax.ShapeDtypeStruct((N, HID), jnp.float32),
                   jax.ShapeDtypeStruct((2, HID), jnp.float32)],
    )(h, st4, rp, rst, g, be, w2, b2, bias, w1n)


def _post1_body(h_ref, st4_ref, rp_ref, rst_ref, g_ref, be_ref, w2_ref,
                b2_ref, bias_ref, out_ref):
    out_ref[...] = _apply_block(h_ref, st4_ref, rp_ref, rst_ref, g_ref,
                                be_ref, w2_ref, b2_ref, bias_ref, DOUT)


def _tc_post1(h, st4, rp, rst, g, be, w2, b2, bias):
    return pl.pallas_call(
        _post1_body,
        grid=(NBLK,),
        in_specs=[pl.BlockSpec((2, NREL, BLK, 32), lambda i: (0, 0, i, 0)),
                  pl.BlockSpec((2, NREL, 2, 32), lambda i: (0, 0, 0, 0)),
                  pl.BlockSpec((BLK, HID), lambda i: (i, 0)),
                  pl.BlockSpec((2, HID), lambda i: (0, 0)),
                  pl.BlockSpec((NREL + 1, HID), lambda i: (0, 0)),
                  pl.BlockSpec((NREL + 1, HID), lambda i: (0, 0)),
                  pl.BlockSpec((NREL + 1, HID, DOUT), lambda i: (0, 0, 0)),
                  pl.BlockSpec((NREL + 1, DOUT), lambda i: (0, 0)),
                  pl.BlockSpec((1, DOUT), lambda i: (0, 0))],
        out_specs=pl.BlockSpec((BLK, DOUT), lambda i: (i, 0)),
        out_shape=jax.ShapeDtypeStruct((N, DOUT), jnp.float32),
    )(h, st4, rp, rst, g, be, w2, b2, bias)


def _edge_body(y2, g2, s2, z, out, idxg, idxs, rows, acc, gsems, ssems, isems):
    cid = lax.axis_index("c")
    sid = lax.axis_index("s")
    base = sid * CHUNKS
    pltpu.sync_copy(z.at[pl.ds(sid * RPT, RPT)], acc.at[pl.ds(sid * RPT, RPT)])
    for p in (0, 1):
        pltpu.async_copy(g2.at[cid, pl.ds(base + p * NBUF, NBUF)], idxg.at[p],
                         isems.at[2 * p])
        pltpu.async_copy(s2.at[pl.ds(base + p * NBUF, NBUF)], idxs.at[p],
                         isems.at[2 * p + 1])
    plsc.subcore_barrier()

    @pl.loop(0, GROUPS // NSLOT)
    def _(it):
        for p in range(NSLOT):
            gi = it * NSLOT + p
            pltpu.make_async_copy(g2.at[cid, pl.ds(base, NBUF)], idxg.at[p],
                                  isems.at[2 * p]).wait()
            pltpu.make_async_copy(s2.at[pl.ds(base, NBUF)], idxs.at[p],
                                  isems.at[2 * p + 1]).wait()
            for b in range(NBUF):
                if p == 0:
                    @pl.when(it > 0)
                    def _():
                        pltpu.make_async_copy(rows.at[b], acc.at[idxs.at[p, b]],
                                              ssems.at[b]).wait()
                else:
                    pltpu.make_async_copy(rows.at[b], acc.at[idxs.at[p, b]],
                                          ssems.at[b]).wait()
                pltpu.async_copy(y2.at[idxg.at[p, b]], rows.at[b], gsems.at[b])

            q = (p + 2) % NSLOT

            @pl.when(gi < GROUPS - 2)
            def _():
                off = base + (gi + 2) * NBUF
                pltpu.async_copy(g2.at[cid, pl.ds(off, NBUF)], idxg.at[q],
                                 isems.at[2 * q])
                pltpu.async_copy(s2.at[pl.ds(off, NBUF)], idxs.at[q],
                                 isems.at[2 * q + 1])
            for b in range(NBUF):
                pltpu.make_async_copy(y2.at[idxg.at[p, b]], rows.at[b],
                                      gsems.at[b]).wait()
                pltpu.async_copy(rows.at[b], acc.at[idxs.at[p, b]], ssems.at[b],
                                 add=True)

    for b in range(NBUF):
        pltpu.make_async_copy(rows.at[b], acc.at[idxs.at[NSLOT - 1, b]],
                              ssems.at[b]).wait()
    plsc.subcore_barrier()
    pltpu.sync_copy(acc.at[pl.ds(sid * WPT, WPT)], out.at[cid, pl.ds(sid * WPT, WPT)])


def _edge_pass(y2, g2, s2, z):
    mesh = plsc.VectorSubcoreMesh(core_axis_name="c", subcore_axis_name="s")
    f = pl.kernel(
        _edge_body,
        out_type=jax.ShapeDtypeStruct((2, WROWS, 32), jnp.float32),
        mesh=mesh,
        scratch_types=[
            pltpu.VMEM((NSLOT, NBUF, C), jnp.int32),
            pltpu.VMEM((NSLOT, NBUF, C), jnp.int32),
            pltpu.VMEM((NBUF, C, 32), jnp.float32),
            pltpu.VMEM_SHARED((ROWS, 32), jnp.float32),
            pltpu.SemaphoreType.DMA((NBUF,)),
            pltpu.SemaphoreType.DMA((NBUF,)),
            pltpu.SemaphoreType.DMA((2 * NSLOT,)),
        ],
        compiler_params=pltpu.CompilerParams(use_tc_tiling_on_sc=False),
    )
    return f(y2, g2, s2, z)


def kernel(x, edge_index, edge_type,
           W1_0, b1_0, g_0, be_0, W2_0, b2_0, bias_0,
           W1_1, b1_1, g_1, be_1, W2_1, b2_1, bias_1):
    pad = E_PAD - E
    zpad = jnp.zeros((pad,), jnp.int32)
    srcp = jnp.concatenate([edge_index[0], zpad]).reshape(EIDX_ROWS, 128)
    dstp = jnp.concatenate([edge_index[1], zpad]).reshape(EIDX_ROWS, 128)
    typp = jnp.concatenate([edge_type, jnp.full((pad,), NREL, jnp.int32)]
                           ).reshape(EIDX_ROWS, 128)
    z = jnp.zeros((ROWS, 32), jnp.float32)

    y2_0, rp0, rst0, g2, s2 = _tc_pre0(x, W1_0, srcp, dstp, typp)
    h0 = _edge_pass(y2_0.reshape(2 * WROWS, 32), g2, s2, z).reshape(2, NREL, N, 32)
    st0 = _tc_stats(h0)
    y2_1, rp1, rst1 = _tc_post0(h0, st0, rp0, rst0, g_0, be_0, W2_0, b2_0,
                                bias_0.reshape(1, HID), W1_1)
    h1 = _edge_pass(y2_1.reshape(2 * WROWS, 32), g2, s2, z).reshape(2, NREL, N, 32)
    st1 = _tc_stats(h1)
    out = _tc_post1(h1, st1, rp1, rst1, g_1, be_1, W2_1, b2_1,
                    bias_1.reshape(1, DOUT))
    return out
